# Initial kernel scaffold; baseline (speedup 1.0000x reference)
import jax
import jax.numpy as jnp
from jax.experimental import pallas as pl


def kernel(x, Wc, bc, gamma, beta):
    raise NotImplementedError("write your pallas kernel here")


# speedup vs baseline: 244.4885x; 244.4885x over previous
import functools

import jax
import jax.numpy as jnp
from jax import lax
from jax.experimental import pallas as pl
from jax.experimental.pallas import tpu as pltpu


def _mr_conv_kernel(xt_ref, xf_ref, wet_ref, wot_ref, bc_ref,
                    y_ref, sums_ref, sumsq_ref,
                    score_scr, max_scr, *, n, c, k_nn):
    b = pl.program_id(0)
    xt = xt_ref[0]
    xf = xf_ref[0]

    sq_row = jnp.sum(xf * xf, axis=0, keepdims=True)
    sq_col = jnp.sum(xt * xt, axis=1, keepdims=True)
    inner = -2.0 * jax.lax.dot_general(
        xt, xf, (((1,), (0,)), ((), ())),
        preferred_element_type=jnp.float32,
        precision=lax.Precision.DEFAULT)
    score_scr[...] = (sq_col + inner) + sq_row
    max_scr[...] = jnp.full((n, c), -jnp.inf, dtype=jnp.float32)

    jidx = lax.broadcasted_iota(jnp.int32, (n, n), 1)

    def body(_, carry):
        s = score_scr[...]
        rowmin = jnp.min(s, axis=1, keepdims=True)
        cand = jnp.where(s <= rowmin, jidx, n)
        minidx = jnp.min(cand, axis=1, keepdims=True)
        sel = jidx == minidx
        onehot = sel.astype(jnp.float32)
        score_scr[...] = jnp.where(sel, jnp.inf, s)
        g = jax.lax.dot_general(
            onehot, xt, (((1,), (0,)), ((), ())),
            preferred_element_type=jnp.float32,
            precision=lax.Precision.HIGHEST)
        max_scr[...] = jnp.maximum(max_scr[...], g)
        return carry

    lax.fori_loop(0, k_nn, body, 0)

    xjmax = max_scr[...] - xt
    y = (jax.lax.dot_general(xt, wet_ref[...], (((1,), (0,)), ((), ())),
                             preferred_element_type=jnp.float32,
                             precision=lax.Precision.HIGHEST)
         + jax.lax.dot_general(xjmax, wot_ref[...], (((1,), (0,)), ((), ())),
                               preferred_element_type=jnp.float32,
                               precision=lax.Precision.HIGHEST)
         + bc_ref[...])
    y_ref[0] = y

    part_s = jnp.sum(y, axis=0, keepdims=True)
    part_q = jnp.sum(y * y, axis=0, keepdims=True)

    @pl.when(b == 0)
    def _():
        sums_ref[...] = part_s
        sumsq_ref[...] = part_q

    @pl.when(b > 0)
    def _():
        sums_ref[...] = sums_ref[...] + part_s
        sumsq_ref[...] = sumsq_ref[...] + part_q


def _bn_relu_kernel(y_ref, sums_ref, sumsq_ref, gamma_ref, beta_ref,
                    out_ref, *, count):
    mean = sums_ref[...] / count
    var = sumsq_ref[...] / count - mean * mean
    inv = 1.0 / jnp.sqrt(var + 1e-5)
    scale = gamma_ref[...] * inv
    shift = beta_ref[...] - mean * scale
    out_ref[0] = jnp.maximum(y_ref[0] * scale + shift, 0.0)


def kernel(x, Wc, bc, gamma, beta):
    b, c, h, w = x.shape
    n = h * w
    o = Wc.shape[0]
    k_nn = 16

    xf = x.reshape(b, c, n)
    xt = jnp.transpose(xf, (0, 2, 1))
    wet = jnp.transpose(Wc[:, 0::2], (1, 0))
    wot = jnp.transpose(Wc[:, 1::2], (1, 0))
    bc2 = bc.reshape(1, o)

    y_raw, sums, sumsq = pl.pallas_call(
        functools.partial(_mr_conv_kernel, n=n, c=c, k_nn=k_nn),
        grid=(b,),
        in_specs=[
            pl.BlockSpec((1, n, c), lambda i: (i, 0, 0)),
            pl.BlockSpec((1, c, n), lambda i: (i, 0, 0)),
            pl.BlockSpec((c, o), lambda i: (0, 0)),
            pl.BlockSpec((c, o), lambda i: (0, 0)),
            pl.BlockSpec((1, o), lambda i: (0, 0)),
        ],
        out_specs=[
            pl.BlockSpec((1, n, o), lambda i: (i, 0, 0)),
            pl.BlockSpec((1, o), lambda i: (0, 0)),
            pl.BlockSpec((1, o), lambda i: (0, 0)),
        ],
        out_shape=[
            jax.ShapeDtypeStruct((b, n, o), jnp.float32),
            jax.ShapeDtypeStruct((1, o), jnp.float32),
            jax.ShapeDtypeStruct((1, o), jnp.float32),
        ],
        scratch_shapes=[
            pltpu.VMEM((n, n), jnp.float32),
            pltpu.VMEM((n, c), jnp.float32),
        ],
    )(xt, xf, wet, wot, bc2)

    out = pl.pallas_call(
        functools.partial(_bn_relu_kernel, count=float(b * n)),
        grid=(b,),
        in_specs=[
            pl.BlockSpec((1, n, o), lambda i: (i, 0, 0)),
            pl.BlockSpec((1, o), lambda i: (0, 0)),
            pl.BlockSpec((1, o), lambda i: (0, 0)),
            pl.BlockSpec((1, o), lambda i: (0, 0)),
            pl.BlockSpec((1, o), lambda i: (0, 0)),
        ],
        out_specs=pl.BlockSpec((1, n, o), lambda i: (i, 0, 0)),
        out_shape=jax.ShapeDtypeStruct((b, n, o), jnp.float32),
    )(y_raw, sums, sumsq, gamma.reshape(1, o), beta.reshape(1, o))

    return jnp.transpose(out, (0, 2, 1)).reshape(b, o, h, w)


# speedup vs baseline: 705.3087x; 2.8848x over previous
import functools

import jax
import jax.numpy as jnp
from jax import lax
from jax.experimental import pallas as pl
from jax.experimental.pallas import tpu as pltpu


def _mr_conv_kernel(xt_ref, xf_ref, wet_ref, wot_ref, bc_ref,
                    y_ref, sums_ref, sumsq_ref,
                    score_scr, max_scr, *, n, c, k_nn):
    b = pl.program_id(0)
    xt = xt_ref[0]
    xf = xf_ref[0]

    sq_row = jnp.sum(xf * xf, axis=0, keepdims=True)
    sq_col = jnp.sum(xt * xt, axis=1, keepdims=True)
    inner = -2.0 * jax.lax.dot_general(
        xt, xf, (((1,), (0,)), ((), ())),
        preferred_element_type=jnp.float32,
        precision=lax.Precision.DEFAULT)
    score_scr[...] = (sq_col + inner) + sq_row
    max_scr[...] = jnp.full((n, c), -jnp.inf, dtype=jnp.float32)

    jidx = lax.broadcasted_iota(jnp.int32, (n, n), 1)

    def body(_, carry):
        s = score_scr[...]
        rowmin = jnp.min(s, axis=1, keepdims=True)
        cand = jnp.where(s <= rowmin, jidx, n)
        minidx = jnp.min(cand, axis=1, keepdims=True)
        sel = jidx == minidx
        onehot = sel.astype(jnp.float32)
        score_scr[...] = jnp.where(sel, jnp.inf, s)
        g = jax.lax.dot_general(
            onehot, xt, (((1,), (0,)), ((), ())),
            preferred_element_type=jnp.float32,
            precision=lax.Precision.DEFAULT)
        max_scr[...] = jnp.maximum(max_scr[...], g)
        return carry

    lax.fori_loop(0, k_nn, body, 0)

    xjmax = max_scr[...] - xt
    y = (jax.lax.dot_general(xt, wet_ref[...], (((1,), (0,)), ((), ())),
                             preferred_element_type=jnp.float32,
                             precision=lax.Precision.DEFAULT)
         + jax.lax.dot_general(xjmax, wot_ref[...], (((1,), (0,)), ((), ())),
                               preferred_element_type=jnp.float32,
                               precision=lax.Precision.DEFAULT)
         + bc_ref[...])
    y_ref[0] = y

    part_s = jnp.sum(y, axis=0, keepdims=True)
    part_q = jnp.sum(y * y, axis=0, keepdims=True)

    @pl.when(b == 0)
    def _():
        sums_ref[...] = part_s
        sumsq_ref[...] = part_q

    @pl.when(b > 0)
    def _():
        sums_ref[...] = sums_ref[...] + part_s
        sumsq_ref[...] = sumsq_ref[...] + part_q


def _bn_relu_kernel(y_ref, sums_ref, sumsq_ref, gamma_ref, beta_ref,
                    out_ref, *, count):
    mean = sums_ref[...] / count
    var = sumsq_ref[...] / count - mean * mean
    inv = 1.0 / jnp.sqrt(var + 1e-5)
    scale = gamma_ref[...] * inv
    shift = beta_ref[...] - mean * scale
    out_ref[0] = jnp.maximum(y_ref[0] * scale + shift, 0.0)


def kernel(x, Wc, bc, gamma, beta):
    b, c, h, w = x.shape
    n = h * w
    o = Wc.shape[0]
    k_nn = 16

    xf = x.reshape(b, c, n)
    xt = jnp.transpose(xf, (0, 2, 1))
    wet = jnp.transpose(Wc[:, 0::2], (1, 0))
    wot = jnp.transpose(Wc[:, 1::2], (1, 0))
    bc2 = bc.reshape(1, o)

    y_raw, sums, sumsq = pl.pallas_call(
        functools.partial(_mr_conv_kernel, n=n, c=c, k_nn=k_nn),
        grid=(b,),
        in_specs=[
            pl.BlockSpec((1, n, c), lambda i: (i, 0, 0)),
            pl.BlockSpec((1, c, n), lambda i: (i, 0, 0)),
            pl.BlockSpec((c, o), lambda i: (0, 0)),
            pl.BlockSpec((c, o), lambda i: (0, 0)),
            pl.BlockSpec((1, o), lambda i: (0, 0)),
        ],
        out_specs=[
            pl.BlockSpec((1, n, o), lambda i: (i, 0, 0)),
            pl.BlockSpec((1, o), lambda i: (0, 0)),
            pl.BlockSpec((1, o), lambda i: (0, 0)),
        ],
        out_shape=[
            jax.ShapeDtypeStruct((b, n, o), jnp.float32),
            jax.ShapeDtypeStruct((1, o), jnp.float32),
            jax.ShapeDtypeStruct((1, o), jnp.float32),
        ],
        scratch_shapes=[
            pltpu.VMEM((n, n), jnp.float32),
            pltpu.VMEM((n, c), jnp.float32),
        ],
    )(xt, xf, wet, wot, bc2)

    out = pl.pallas_call(
        functools.partial(_bn_relu_kernel, count=float(b * n)),
        grid=(b,),
        in_specs=[
            pl.BlockSpec((1, n, o), lambda i: (i, 0, 0)),
            pl.BlockSpec((1, o), lambda i: (0, 0)),
            pl.BlockSpec((1, o), lambda i: (0, 0)),
            pl.BlockSpec((1, o), lambda i: (0, 0)),
            pl.BlockSpec((1, o), lambda i: (0, 0)),
        ],
        out_specs=pl.BlockSpec((1, n, o), lambda i: (i, 0, 0)),
        out_shape=jax.ShapeDtypeStruct((b, n, o), jnp.float32),
    )(y_raw, sums, sumsq, gamma.reshape(1, o), beta.reshape(1, o))

    return jnp.transpose(out, (0, 2, 1)).reshape(b, o, h, w)
